# baseline (speedup 1.0000x reference)
import functools

import jax
import jax.numpy as jnp
from jax import lax
from jax.experimental import pallas as pl
from jax.experimental.pallas import tpu as pltpu
from jax.experimental.pallas import tpu_sc as plsc

VOCAB = 30522
HIDDEN = 1024
MAX_POS = 2048
B, S = 4, 2048
NTOK = B * S
EPS = 1e-12

NC, NS = 2, 16
NW = NC * NS
TPW = NTOK // NW
CHUNK = 32
NCHUNK = TPW // CHUNK

_sc_mesh = plsc.VectorSubcoreMesh(core_axis_name="c", subcore_axis_name="s")


@functools.partial(
    pl.kernel,
    out_type=jax.ShapeDtypeStruct((NTOK, HIDDEN), jnp.float32),
    mesh=_sc_mesh,
    scratch_types=[
        pltpu.VMEM((TPW,), jnp.int32),
        pltpu.VMEM((CHUNK, HIDDEN), jnp.float32),
        pltpu.VMEM((CHUNK, HIDDEN), jnp.float32),
        pltpu.SemaphoreType.DMA,
        pltpu.SemaphoreType.DMA,
        pltpu.SemaphoreType.DMA,
        pltpu.SemaphoreType.DMA,
    ],
)
def _sc_gather(idx_hbm, table_hbm, out_hbm, idx_v, rows0, rows1, g0, g1, s0, s1):
    wid = lax.axis_index("s") * NC + lax.axis_index("c")
    base = wid * TPW
    pltpu.sync_copy(idx_hbm.at[pl.ds(base, TPW)], idx_v)
    bufs, gsems, ssems = [rows0, rows1], [g0, g1], [s0, s1]
    gather_h = [None, None]
    scatter_h = [None, None]
    for ci in range(NCHUNK + 1):
        b = ci % 2
        if ci < NCHUNK:
            if scatter_h[b] is not None:
                scatter_h[b].wait()
            gather_h[b] = pltpu.async_copy(
                table_hbm.at[idx_v.at[pl.ds(ci * CHUNK, CHUNK)]], bufs[b], gsems[b]
            )
        if ci > 0:
            pb = (ci - 1) % 2
            gather_h[pb].wait()
            scatter_h[pb] = pltpu.async_copy(
                bufs[pb], out_hbm.at[pl.ds(base + (ci - 1) * CHUNK, CHUNK)], ssems[pb]
            )
    scatter_h[(NCHUNK - 2) % 2].wait()
    scatter_h[(NCHUNK - 1) % 2].wait()


TOK_BLK = 256
SMALL_ROWS = 128


def _tc_body(gath_ref, pos_ref, hi_ref, lo_ref, tt_ref, pk_ref, dp_ref,
             gamma_ref, beta_ref, out_ref):
    tt = tt_ref[0, 0]
    pk = pk_ref[0, 0]
    dp = dp_ref[0, 0]
    col = lax.broadcasted_iota(jnp.int32, (TOK_BLK, SMALL_ROWS), 1)
    m = ((col == tt) | (col == pk + 8) | (col == dp + 64)).astype(jnp.bfloat16)
    small = (jnp.dot(m, hi_ref[...], preferred_element_type=jnp.float32)
             + jnp.dot(m, lo_ref[...], preferred_element_type=jnp.float32)
             * jnp.float32(1.0 / 256.0))
    x = gath_ref[0, 0] + pos_ref[0] + small
    mean = jnp.mean(x, axis=-1, keepdims=True)
    xm = x - mean
    var = jnp.mean(xm * xm, axis=-1, keepdims=True)
    y = xm * lax.rsqrt(var + EPS)
    out_ref[0, 0] = y * gamma_ref[...] + beta_ref[...]


SBLK = S // TOK_BLK

_tc_fused = pl.pallas_call(
    _tc_body,
    grid=(SBLK, B),
    in_specs=[
        pl.BlockSpec((1, 1, TOK_BLK, HIDDEN), lambda i, j: (j, i, 0, 0)),
        pl.BlockSpec((1, TOK_BLK, HIDDEN), lambda i, j: (i, 0, 0)),
        pl.BlockSpec((SMALL_ROWS, HIDDEN), lambda i, j: (0, 0)),
        pl.BlockSpec((SMALL_ROWS, HIDDEN), lambda i, j: (0, 0)),
        pl.BlockSpec((1, 1, TOK_BLK, 1), lambda i, j: (j, i, 0, 0)),
        pl.BlockSpec((1, 1, TOK_BLK, 1), lambda i, j: (j, i, 0, 0)),
        pl.BlockSpec((1, 1, TOK_BLK, 1), lambda i, j: (j, i, 0, 0)),
        pl.BlockSpec((1, HIDDEN), lambda i, j: (0, 0)),
        pl.BlockSpec((1, HIDDEN), lambda i, j: (0, 0)),
    ],
    out_specs=pl.BlockSpec((1, 1, TOK_BLK, HIDDEN), lambda i, j: (j, i, 0, 0)),
    out_shape=jax.ShapeDtypeStruct((B, SBLK, TOK_BLK, HIDDEN), jnp.float32),
)


def kernel(input_ids, token_type_ids, pos_ids, dep_ids,
           W_word, W_position, W_type, W_tokpos, W_dep, ln_gamma, ln_beta):
    idx = input_ids.reshape(-1).astype(jnp.int32)
    gathered = _sc_gather(idx, W_word)

    gath4 = gathered.reshape(B, SBLK, TOK_BLK, HIDDEN)
    pos3 = W_position.reshape(SBLK, TOK_BLK, HIDDEN)
    tt = token_type_ids.reshape(B, SBLK, TOK_BLK, 1).astype(jnp.int32)
    pk = pos_ids.reshape(B, SBLK, TOK_BLK, 1).astype(jnp.int32)
    dp = dep_ids.reshape(B, SBLK, TOK_BLK, 1).astype(jnp.int32)
    zeros = lambda n: jnp.zeros((n, HIDDEN), jnp.float32)
    small = jnp.concatenate([W_type, zeros(6), W_tokpos, zeros(6), W_dep], axis=0)
    hi = small.astype(jnp.bfloat16)
    lo = ((small - hi.astype(jnp.float32)) * 256.0).astype(jnp.bfloat16)
    out = _tc_fused(gath4, pos3, hi, lo, tt, pk, dp,
                    ln_gamma.reshape(1, HIDDEN), ln_beta.reshape(1, HIDDEN))
    return out.reshape(B, S, HIDDEN)


# speedup vs baseline: 1.0278x; 1.0278x over previous
import functools

import jax
import jax.numpy as jnp
from jax import lax
from jax.experimental import pallas as pl
from jax.experimental.pallas import tpu as pltpu
from jax.experimental.pallas import tpu_sc as plsc

VOCAB = 30522
HIDDEN = 1024
MAX_POS = 2048
B, S = 4, 2048
NTOK = B * S
EPS = 1e-12

NC, NS = 2, 16
NW = NC * NS
TPW = NTOK // NW
CHUNK = 32
NCHUNK = TPW // CHUNK

_sc_mesh = plsc.VectorSubcoreMesh(core_axis_name="c", subcore_axis_name="s")


@functools.partial(
    pl.kernel,
    out_type=jax.ShapeDtypeStruct((NTOK, HIDDEN), jnp.float32),
    mesh=_sc_mesh,
    scratch_types=[
        pltpu.VMEM((TPW,), jnp.int32),
        pltpu.VMEM((CHUNK, HIDDEN), jnp.float32),
        pltpu.VMEM((CHUNK, HIDDEN), jnp.float32),
        pltpu.SemaphoreType.DMA,
        pltpu.SemaphoreType.DMA,
        pltpu.SemaphoreType.DMA,
        pltpu.SemaphoreType.DMA,
    ],
)
def _sc_gather(idx_hbm, table_hbm, out_hbm, idx_v, rows0, rows1, g0, g1, s0, s1):
    wid = lax.axis_index("s") * NC + lax.axis_index("c")
    base = wid * TPW
    pltpu.sync_copy(idx_hbm.at[pl.ds(base, TPW)], idx_v)
    bufs, gsems, ssems = [rows0, rows1], [g0, g1], [s0, s1]
    gather_h = [None, None]
    scatter_h = [None, None]
    for ci in range(NCHUNK + 1):
        b = ci % 2
        if ci < NCHUNK:
            if scatter_h[b] is not None:
                scatter_h[b].wait()
            gather_h[b] = pltpu.async_copy(
                table_hbm.at[idx_v.at[pl.ds(ci * CHUNK, CHUNK)]], bufs[b], gsems[b]
            )
        if ci > 0:
            pb = (ci - 1) % 2
            gather_h[pb].wait()
            scatter_h[pb] = pltpu.async_copy(
                bufs[pb], out_hbm.at[pl.ds(base + (ci - 1) * CHUNK, CHUNK)], ssems[pb]
            )
    scatter_h[(NCHUNK - 2) % 2].wait()
    scatter_h[(NCHUNK - 1) % 2].wait()


TOK_BLK = 256
SMALL_ROWS = 128


def _tc_body(gath_ref, pos_ref, hi_ref, tt_ref, pk_ref, dp_ref,
             gamma_ref, beta_ref, out_ref):
    tt = tt_ref[0, 0]
    pk = pk_ref[0, 0]
    dp = dp_ref[0, 0]
    col = lax.broadcasted_iota(jnp.int32, (TOK_BLK, SMALL_ROWS), 1)
    m = ((col == tt) | (col == pk + 8) | (col == dp + 64)).astype(jnp.bfloat16)
    small = jnp.dot(m, hi_ref[...], preferred_element_type=jnp.float32)
    x = gath_ref[0, 0] + pos_ref[0] + small
    mean = jnp.mean(x, axis=-1, keepdims=True)
    xm = x - mean
    var = jnp.mean(xm * xm, axis=-1, keepdims=True)
    y = xm * lax.rsqrt(var + EPS)
    out_ref[0, 0] = y * gamma_ref[...] + beta_ref[...]


SBLK = S // TOK_BLK

_tc_fused = pl.pallas_call(
    _tc_body,
    grid=(SBLK, B),
    in_specs=[
        pl.BlockSpec((1, 1, TOK_BLK, HIDDEN), lambda i, j: (j, i, 0, 0)),
        pl.BlockSpec((1, TOK_BLK, HIDDEN), lambda i, j: (i, 0, 0)),
        pl.BlockSpec((SMALL_ROWS, HIDDEN), lambda i, j: (0, 0)),
        pl.BlockSpec((1, 1, TOK_BLK, 1), lambda i, j: (j, i, 0, 0)),
        pl.BlockSpec((1, 1, TOK_BLK, 1), lambda i, j: (j, i, 0, 0)),
        pl.BlockSpec((1, 1, TOK_BLK, 1), lambda i, j: (j, i, 0, 0)),
        pl.BlockSpec((1, HIDDEN), lambda i, j: (0, 0)),
        pl.BlockSpec((1, HIDDEN), lambda i, j: (0, 0)),
    ],
    out_specs=pl.BlockSpec((1, 1, TOK_BLK, HIDDEN), lambda i, j: (j, i, 0, 0)),
    out_shape=jax.ShapeDtypeStruct((B, SBLK, TOK_BLK, HIDDEN), jnp.float32),
)


def kernel(input_ids, token_type_ids, pos_ids, dep_ids,
           W_word, W_position, W_type, W_tokpos, W_dep, ln_gamma, ln_beta):
    idx = input_ids.reshape(-1).astype(jnp.int32)
    gathered = _sc_gather(idx, W_word)

    gath4 = gathered.reshape(B, SBLK, TOK_BLK, HIDDEN)
    pos3 = W_position.reshape(SBLK, TOK_BLK, HIDDEN)
    tt = token_type_ids.reshape(B, SBLK, TOK_BLK, 1).astype(jnp.int32)
    pk = pos_ids.reshape(B, SBLK, TOK_BLK, 1).astype(jnp.int32)
    dp = dep_ids.reshape(B, SBLK, TOK_BLK, 1).astype(jnp.int32)
    zeros = lambda n: jnp.zeros((n, HIDDEN), jnp.float32)
    small = jnp.concatenate([W_type, zeros(6), W_tokpos, zeros(6), W_dep], axis=0)
    hi = small.astype(jnp.bfloat16)
    out = _tc_fused(gath4, pos3, hi, tt, pk, dp,
                    ln_gamma.reshape(1, HIDDEN), ln_beta.reshape(1, HIDDEN))
    return out.reshape(B, S, HIDDEN)
